# Initial kernel scaffold; baseline (speedup 1.0000x reference)
import jax
import jax.numpy as jnp
from jax.experimental import pallas as pl


def kernel(x, node_depth, attr2vocab):
    raise NotImplementedError("write your pallas kernel here")


# speedup vs baseline: 4.2553x; 4.2553x over previous
import functools

import jax
import jax.numpy as jnp
from jax import lax
from jax.experimental import pallas as pl
from jax.experimental.pallas import tpu as pltpu
from jax.experimental.pallas import tpu_sc as plsc

NUM_VOCAB = 5008
MAX_SEQ_LEN = 5
MIN_OCC = 2
NODES_PER_GRAPH = 32
VOCAB_BITS = 13
VOCAB_MASK = (1 << VOCAB_BITS) - 1


def _sc_packed_ranks(xcol, attr2vocab, num_graphs):
    info = plsc.get_sparse_core_info()
    NC, NS, L = info.num_cores, info.num_subcores, info.num_lanes
    NW = NC * NS
    graphs_per_w = num_graphs // NW
    nodes_per_w = graphs_per_w * NODES_PER_GRAPH
    n = num_graphs * NODES_PER_GRAPH

    mesh = plsc.VectorSubcoreMesh(core_axis_name="c", subcore_axis_name="s")

    @functools.partial(
        pl.kernel,
        mesh=mesh,
        out_type=jax.ShapeDtypeStruct((n,), jnp.int32),
        scratch_types=[
            pltpu.VMEM((nodes_per_w,), jnp.int32),
            pltpu.VMEM((nodes_per_w,), jnp.int32),
            pltpu.VMEM((nodes_per_w,), jnp.int32),
            pltpu.SemaphoreType.DMA,
        ],
    )
    def k(xcol_hbm, tbl_hbm, out_hbm, xv, vv, pk_v, sem):
        wid = lax.axis_index("s") * NC + lax.axis_index("c")
        base_node = wid * nodes_per_w
        pltpu.sync_copy(xcol_hbm.at[pl.ds(base_node, nodes_per_w)], xv)

        CH = 128
        copies = [
            pltpu.async_copy(
                tbl_hbm.at[xv.at[pl.ds(c * CH, CH)]],
                vv.at[pl.ds(c * CH, CH)],
                sem,
            )
            for c in range(nodes_per_w // CH)
        ]
        for cp in copies:
            cp.wait()

        lanes = lax.iota(jnp.int32, L)

        def graph_body(g, carry):
            gbase = g * NODES_PER_GRAPH
            a = vv[pl.ds(gbase, L)]
            b = vv[pl.ds(gbase + L, L)]

            zeros = jnp.zeros((L,), jnp.int32)
            big = jnp.full((L,), 2 * NODES_PER_GRAPH, jnp.int32)
            cnt_a, cnt_b, first_a, first_b = zeros, zeros, big, big
            for e in range(NODES_PER_GRAPH):
                sv = a[e] if e < L else b[e - L]
                ma = a == sv
                mb = b == sv
                cnt_a = cnt_a + jnp.where(ma, 1, 0)
                cnt_b = cnt_b + jnp.where(mb, 1, 0)
                first_a = jnp.minimum(first_a, jnp.where(ma, e, 2 * NODES_PER_GRAPH))
                first_b = jnp.minimum(first_b, jnp.where(mb, e, 2 * NODES_PER_GRAPH))

            va = jnp.logical_and(first_a == lanes, a != NUM_VOCAB)
            vb = jnp.logical_and(first_b == lanes + L, b != NUM_VOCAB)
            ka = jnp.where(va, (cnt_a << VOCAB_BITS) + (VOCAB_MASK - a), 0)
            kb = jnp.where(vb, (cnt_b << VOCAB_BITS) + (VOCAB_MASK - b), 0)

            rank_a, rank_b = zeros, zeros
            for e in range(NODES_PER_GRAPH):
                sk = ka[e] if e < L else kb[e - L]
                rank_a = rank_a + jnp.where(sk > ka, 1, 0)
                rank_b = rank_b + jnp.where(sk > kb, 1, 0)

            min_key = MIN_OCC << VOCAB_BITS
            pa = jnp.where(
                jnp.logical_and(ka >= min_key, rank_a < MAX_SEQ_LEN),
                (rank_a << VOCAB_BITS) + (VOCAB_MASK - (ka & VOCAB_MASK)) + 1,
                0,
            )
            pb = jnp.where(
                jnp.logical_and(kb >= min_key, rank_b < MAX_SEQ_LEN),
                (rank_b << VOCAB_BITS) + (VOCAB_MASK - (kb & VOCAB_MASK)) + 1,
                0,
            )
            pk_v[pl.ds(gbase, L)] = pa
            pk_v[pl.ds(gbase + L, L)] = pb
            return carry

        lax.fori_loop(0, graphs_per_w, graph_body, 0)
        pltpu.sync_copy(pk_v, out_hbm.at[pl.ds(base_node, nodes_per_w)])

    return k(xcol, attr2vocab)


def _tc_emit_preds(packed, num_graphs):
    BR = 128
    nb = num_graphs // BR

    def body(pk_ref, out_ref):
        p = pk_ref[...] - 1
        rank = p >> VOCAB_BITS
        tok1 = (p & VOCAB_MASK) + 1
        cols = lax.broadcasted_iota(jnp.int32, (BR, NUM_VOCAB), 1)
        for j in range(MAX_SEQ_LEN):
            tj = jnp.sum(jnp.where(rank == j, tok1, 0), axis=1, keepdims=True) - 1
            out_ref[j] = jnp.where(cols == tj, 1, -1)

    return pl.pallas_call(
        body,
        grid=(nb,),
        in_specs=[pl.BlockSpec((BR, NODES_PER_GRAPH), lambda i: (i, 0))],
        out_specs=pl.BlockSpec((MAX_SEQ_LEN, BR, NUM_VOCAB), lambda i: (0, i, 0)),
        out_shape=jax.ShapeDtypeStruct(
            (MAX_SEQ_LEN, num_graphs, NUM_VOCAB), jnp.int32
        ),
    )(packed)


def kernel(x, node_depth, attr2vocab):
    num_graphs = node_depth.shape[0] // NODES_PER_GRAPH
    xcol = x[:, 1]
    packed = _sc_packed_ranks(xcol, attr2vocab, num_graphs)
    packed = packed.reshape(num_graphs, NODES_PER_GRAPH)
    preds = _tc_emit_preds(packed, num_graphs)
    return tuple(preds[j] for j in range(MAX_SEQ_LEN))


# speedup vs baseline: 5.8334x; 1.3709x over previous
import functools

import jax
import jax.numpy as jnp
from jax import lax
from jax.experimental import pallas as pl
from jax.experimental.pallas import tpu as pltpu
from jax.experimental.pallas import tpu_sc as plsc

NUM_VOCAB = 5008
MAX_SEQ_LEN = 5
MIN_OCC = 2
NODES_PER_GRAPH = 32
VOCAB_BITS = 13
VOCAB_MASK = (1 << VOCAB_BITS) - 1


def _sc_packed_ranks(xcol, attr2vocab, num_graphs):
    info = plsc.get_sparse_core_info()
    NC, NS, L = info.num_cores, info.num_subcores, info.num_lanes
    NW = NC * NS
    graphs_per_w = num_graphs // NW
    nodes_per_w = graphs_per_w * NODES_PER_GRAPH
    n = num_graphs * NODES_PER_GRAPH

    mesh = plsc.VectorSubcoreMesh(core_axis_name="c", subcore_axis_name="s")

    @functools.partial(
        pl.kernel,
        mesh=mesh,
        out_type=jax.ShapeDtypeStruct((n,), jnp.int32),
        scratch_types=[
            pltpu.VMEM((nodes_per_w,), jnp.int32),
            pltpu.VMEM((nodes_per_w,), jnp.int32),
            pltpu.VMEM((nodes_per_w,), jnp.int32),
            pltpu.SemaphoreType.DMA,
        ],
    )
    def k(xcol_hbm, tbl_hbm, out_hbm, xv, vv, pk_v, sem):
        wid = lax.axis_index("s") * NC + lax.axis_index("c")
        base_node = wid * nodes_per_w
        pltpu.sync_copy(xcol_hbm.at[pl.ds(base_node, nodes_per_w)], xv)

        CH = 128
        copies = [
            pltpu.async_copy(
                tbl_hbm.at[xv.at[pl.ds(c * CH, CH)]],
                vv.at[pl.ds(c * CH, CH)],
                sem,
            )
            for c in range(nodes_per_w // CH)
        ]
        for cp in copies:
            cp.wait()

        lanes = lax.iota(jnp.int32, L)

        def graph_body(g, carry):
            gbase = g * NODES_PER_GRAPH
            a = vv[pl.ds(gbase, L)]
            b = vv[pl.ds(gbase + L, L)]

            zeros = jnp.zeros((L,), jnp.int32)
            big = jnp.full((L,), 2 * NODES_PER_GRAPH, jnp.int32)
            cnt_a, cnt_b, first_a, first_b = zeros, zeros, big, big
            for e in range(NODES_PER_GRAPH):
                sv = a[e] if e < L else b[e - L]
                ma = a == sv
                mb = b == sv
                cnt_a = cnt_a + jnp.where(ma, 1, 0)
                cnt_b = cnt_b + jnp.where(mb, 1, 0)
                first_a = jnp.minimum(first_a, jnp.where(ma, e, 2 * NODES_PER_GRAPH))
                first_b = jnp.minimum(first_b, jnp.where(mb, e, 2 * NODES_PER_GRAPH))

            va = jnp.logical_and(first_a == lanes, a != NUM_VOCAB)
            vb = jnp.logical_and(first_b == lanes + L, b != NUM_VOCAB)
            ka = jnp.where(va, (cnt_a << VOCAB_BITS) + (VOCAB_MASK - a), 0)
            kb = jnp.where(vb, (cnt_b << VOCAB_BITS) + (VOCAB_MASK - b), 0)

            rank_a, rank_b = zeros, zeros
            for e in range(NODES_PER_GRAPH):
                sk = ka[e] if e < L else kb[e - L]
                rank_a = rank_a + jnp.where(sk > ka, 1, 0)
                rank_b = rank_b + jnp.where(sk > kb, 1, 0)

            min_key = MIN_OCC << VOCAB_BITS
            pa = jnp.where(
                jnp.logical_and(ka >= min_key, rank_a < MAX_SEQ_LEN),
                (rank_a << VOCAB_BITS) + (VOCAB_MASK - (ka & VOCAB_MASK)) + 1,
                0,
            )
            pb = jnp.where(
                jnp.logical_and(kb >= min_key, rank_b < MAX_SEQ_LEN),
                (rank_b << VOCAB_BITS) + (VOCAB_MASK - (kb & VOCAB_MASK)) + 1,
                0,
            )
            pk_v[pl.ds(gbase, L)] = pa
            pk_v[pl.ds(gbase + L, L)] = pb
            return carry

        lax.fori_loop(0, graphs_per_w, graph_body, 0)
        pltpu.sync_copy(pk_v, out_hbm.at[pl.ds(base_node, nodes_per_w)])

    return k(xcol, attr2vocab)


def _tc_emit_preds(packed, num_graphs):
    BR = 128
    nb = num_graphs // BR

    def body(pk_ref, *out_refs):
        p = pk_ref[...] - 1
        rank = p >> VOCAB_BITS
        tok1 = (p & VOCAB_MASK) + 1
        cols = lax.broadcasted_iota(jnp.int32, (BR, NUM_VOCAB), 1)
        for j in range(MAX_SEQ_LEN):
            tj = jnp.sum(jnp.where(rank == j, tok1, 0), axis=1, keepdims=True) - 1
            out_refs[j][...] = jnp.where(cols == tj, 1, -1)

    return pl.pallas_call(
        body,
        grid=(nb,),
        in_specs=[pl.BlockSpec((BR, NODES_PER_GRAPH), lambda i: (i, 0))],
        out_specs=[
            pl.BlockSpec((BR, NUM_VOCAB), lambda i: (i, 0))
            for _ in range(MAX_SEQ_LEN)
        ],
        out_shape=[
            jax.ShapeDtypeStruct((num_graphs, NUM_VOCAB), jnp.int32)
            for _ in range(MAX_SEQ_LEN)
        ],
    )(packed)


def kernel(x, node_depth, attr2vocab):
    num_graphs = node_depth.shape[0] // NODES_PER_GRAPH
    xcol = x[:, 1]
    packed = _sc_packed_ranks(xcol, attr2vocab, num_graphs)
    packed = packed.reshape(num_graphs, NODES_PER_GRAPH)
    return tuple(_tc_emit_preds(packed, num_graphs))
